# baseline (speedup 1.0000x reference)
import functools

import jax
import jax.numpy as jnp
from jax import lax
from jax.experimental import pallas as pl
from jax.experimental.pallas import tpu as pltpu
from jax.experimental.pallas import tpu_sc as plsc

VOCAB = 100000
EMB = 128
HID = 512
B = 4096
L = 200
EPS = 1e-5

CHUNK0 = 120
CHUNK1 = L - CHUNK0
VREGS = EMB // 16
NSLOT = 4
BLK = 16


@functools.lru_cache(maxsize=None)
def _sc_pool():
    info = plsc.get_sparse_core_info()
    nc, ns = info.num_cores, info.num_subcores
    nw = nc * ns
    spw = B // nw
    nblk = spw // BLK

    mesh = plsc.VectorSubcoreMesh(core_axis_name="c", subcore_axis_name="s")

    @functools.partial(
        pl.kernel,
        mesh=mesh,
        out_type=jax.ShapeDtypeStruct((B * EMB,), jnp.float32),
        scratch_types=[
            pltpu.VMEM((2 * BLK * L,), jnp.int32),
            pltpu.VMEM((NSLOT * L, EMB), jnp.float32),
            pltpu.VMEM((spw * EMB,), jnp.float32),
            pltpu.SemaphoreType.DMA,
            pltpu.SemaphoreType.DMA,
            pltpu.SemaphoreType.DMA,
            pltpu.SemaphoreType.DMA,
            pltpu.SemaphoreType.DMA,
        ],
    )
    def pool(x_hbm, emb_hbm, out_hbm, idx_v, rows_v, out_v,
             sem0, sem1, sem2, sem3, semi):
        sems = (sem0, sem1, sem2, sem3)
        c = lax.axis_index("c")
        s = lax.axis_index("s")
        wid = s * nc + c
        xbase = wid * (spw * L)

        def idx_off(i):
            blk = i // BLK
            par = lax.rem(blk, 2)
            loc = lax.rem(i, BLK)
            return par * (BLK * L) + loc * L

        def issue(i, slot):
            o = idx_off(i)
            sem = sems[slot]
            pltpu.async_copy(
                emb_hbm.at[idx_v.at[pl.ds(o, CHUNK0)]],
                rows_v.at[pl.ds(slot * L, CHUNK0)], sem)
            pltpu.async_copy(
                emb_hbm.at[idx_v.at[pl.ds(o + CHUNK0, CHUNK1)]],
                rows_v.at[pl.ds(slot * L + CHUNK0, CHUNK1)], sem)

        def drain(i, slot):
            o = idx_off(i)
            sem = sems[slot]
            pltpu.make_async_copy(
                emb_hbm.at[idx_v.at[pl.ds(o, CHUNK0)]],
                rows_v.at[pl.ds(slot * L, CHUNK0)], sem).wait()
            pltpu.make_async_copy(
                emb_hbm.at[idx_v.at[pl.ds(o + CHUNK0, CHUNK1)]],
                rows_v.at[pl.ds(slot * L + CHUNK0, CHUNK1)], sem).wait()

        def reduce_store(i, slot):
            unroll = 8

            def body(jj, acc):
                j0 = jj * unroll
                for u in range(unroll):
                    acc = tuple(
                        acc[k] + rows_v[slot * L + j0 + u, pl.ds(16 * k, 16)]
                        for k in range(VREGS))
                return acc

            acc = tuple(jnp.zeros((16,), jnp.float32) for _ in range(VREGS))
            acc = lax.fori_loop(0, L // unroll, body, acc)
            for k in range(VREGS):
                out_v[pl.ds(i * EMB + 16 * k, 16)] = acc[k]

        def quad(i0):
            issue(i0 + 3, 3)
            for u in range(3):
                drain(i0 + u, u)
                reduce_store(i0 + u, u)

                @pl.when(i0 + 4 + u < spw)
                def _(u=u):
                    issue(i0 + 4 + u, u)

            drain(i0 + 3, 3)
            reduce_store(i0 + 3, 3)

        def idx_copy(b, sem):
            return pltpu.make_async_copy(
                x_hbm.at[pl.ds(xbase + b * (BLK * L), BLK * L)],
                idx_v.at[pl.ds(lax.rem(b, 2) * (BLK * L), BLK * L)], sem)

        pltpu.sync_copy(x_hbm.at[pl.ds(xbase, BLK * L)],
                        idx_v.at[pl.ds(0, BLK * L)])
        for i in range(NSLOT - 1):
            issue(i, i)

        def block(b, carry):
            @pl.when(b + 1 < nblk)
            def _():
                idx_copy(b + 1, semi).start()

            for qq in range(3):
                quad(b * BLK + 4 * qq)

            @pl.when(b + 1 < nblk)
            def _():
                idx_copy(b + 1, semi).wait()

            quad(b * BLK + 12)
            return carry

        lax.fori_loop(0, nblk, block, 0)
        pltpu.sync_copy(out_v, out_hbm.at[pl.ds(wid * (spw * EMB), spw * EMB)])

    return pool


def _dense_body(e_ref, w_ref, b_ref, g_ref, bt_ref, out_ref):
    e = e_ref[...]
    w = w_ref[...]
    z = lax.dot_general(e, w, (((1,), (1,)), ((), ())),
                        preferred_element_type=jnp.float32)
    h = jax.nn.sigmoid(z * (1.0 / L) + b_ref[...])
    mu = jnp.mean(h, axis=0, keepdims=True)
    var = jnp.mean((h - mu) ** 2, axis=0, keepdims=True)
    out_ref[...] = (h - mu) * lax.rsqrt(var + EPS) * g_ref[...] + bt_ref[...]


def _tc_dense(e_sum, w_h, b_h, gamma, beta):
    return pl.pallas_call(
        _dense_body,
        out_shape=jax.ShapeDtypeStruct((B, HID), jnp.float32),
    )(e_sum, w_h, b_h.reshape(1, HID), gamma.reshape(1, HID),
      beta.reshape(1, HID))


def kernel(x, emb, W_h, b_h, gamma, beta):
    x = x.astype(jnp.int32).reshape(B * L)
    e_sum = _sc_pool()(x, emb).reshape(B, EMB)
    return _tc_dense(e_sum, W_h, b_h, gamma, beta)


# speedup vs baseline: 1.0200x; 1.0200x over previous
import functools

import jax
import jax.numpy as jnp
from jax import lax
from jax.experimental import pallas as pl
from jax.experimental.pallas import tpu as pltpu
from jax.experimental.pallas import tpu_sc as plsc

VOCAB = 100000
EMB = 128
HID = 512
B = 4096
L = 200
EPS = 1e-5

CHUNKS = ((0, 64), (64, 64), (128, 72))
VREGS = EMB // 16


@functools.lru_cache(maxsize=None)
def _sc_pool():
    info = plsc.get_sparse_core_info()
    nc, ns = info.num_cores, info.num_subcores
    nw = nc * ns
    spw = B // nw

    mesh = plsc.VectorSubcoreMesh(core_axis_name="c", subcore_axis_name="s")

    @functools.partial(
        pl.kernel,
        mesh=mesh,
        out_type=jax.ShapeDtypeStruct((B * EMB,), jnp.float32),
        scratch_types=[
            pltpu.VMEM((spw * L,), jnp.int32),
            pltpu.VMEM((3 * L, EMB), jnp.float32),
            pltpu.VMEM((spw * EMB,), jnp.float32),
            pltpu.SemaphoreType.DMA,
            pltpu.SemaphoreType.DMA,
            pltpu.SemaphoreType.DMA,
        ],
    )
    def pool(x_hbm, emb_hbm, out_hbm, idx_v, rows_v, out_v, sem0, sem1, sem2):
        c = lax.axis_index("c")
        s = lax.axis_index("s")
        wid = s * nc + c
        pltpu.sync_copy(x_hbm.at[pl.ds(wid * (spw * L), spw * L)], idx_v)

        def issue(i, slot, sem):
            for off, ln in CHUNKS:
                pltpu.async_copy(
                    emb_hbm.at[idx_v.at[pl.ds(i * L + off, ln)]],
                    rows_v.at[pl.ds(slot * L + off, ln)], sem)

        def drain(i, slot, sem):
            for off, ln in CHUNKS:
                pltpu.make_async_copy(
                    emb_hbm.at[idx_v.at[pl.ds(i * L + off, ln)]],
                    rows_v.at[pl.ds(slot * L + off, ln)], sem).wait()

        def reduce_store(i, slot):
            UNROLL = 8

            def body(jj, acc):
                j0 = jj * UNROLL
                for u in range(UNROLL):
                    acc = tuple(
                        acc[k] + rows_v[slot * L + j0 + u, pl.ds(16 * k, 16)]
                        for k in range(VREGS))
                return acc

            acc = tuple(jnp.zeros((16,), jnp.float32) for _ in range(VREGS))
            acc = lax.fori_loop(0, L // UNROLL, body, acc)
            for k in range(VREGS):
                out_v[pl.ds(i * EMB + 16 * k, 16)] = acc[k]

        issue(0, 0, sem0)
        issue(1, 1, sem1)

        def triple(t, carry):
            i0 = 3 * t
            issue(i0 + 2, 2, sem2)
            drain(i0, 0, sem0)
            reduce_store(i0, 0)

            @pl.when(i0 + 3 < spw)
            def _():
                issue(i0 + 3, 0, sem0)

            drain(i0 + 1, 1, sem1)
            reduce_store(i0 + 1, 1)

            @pl.when(i0 + 4 < spw)
            def _():
                issue(i0 + 4, 1, sem1)

            drain(i0 + 2, 2, sem2)
            reduce_store(i0 + 2, 2)
            return carry

        ntrip = spw // 3
        lax.fori_loop(0, ntrip, triple, 0)
        rem = spw - 3 * ntrip
        epi = ((3 * ntrip, 0, sem0), (3 * ntrip + 1, 1, sem1),
               (3 * ntrip + 2, 2, sem2))[:rem]
        for i, slot, sem in epi:
            drain(i, slot, sem)
            reduce_store(i, slot)
        pltpu.sync_copy(out_v, out_hbm.at[pl.ds(wid * (spw * EMB), spw * EMB)])

    return pool


def _dense_body(e_ref, w_ref, b_ref, g_ref, bt_ref, out_ref):
    e = e_ref[...]
    w = w_ref[...]
    z = lax.dot_general(e, w, (((1,), (1,)), ((), ())),
                        preferred_element_type=jnp.float32)
    h = jax.nn.sigmoid(z * (1.0 / L) + b_ref[...])
    mu = jnp.mean(h, axis=0, keepdims=True)
    var = jnp.mean((h - mu) ** 2, axis=0, keepdims=True)
    out_ref[...] = (h - mu) * lax.rsqrt(var + EPS) * g_ref[...] + bt_ref[...]


def _tc_dense(e_sum, w_h, b_h, gamma, beta):
    return pl.pallas_call(
        _dense_body,
        out_shape=jax.ShapeDtypeStruct((B, HID), jnp.float32),
    )(e_sum, w_h, b_h.reshape(1, HID), gamma.reshape(1, HID),
      beta.reshape(1, HID))


def kernel(x, emb, W_h, b_h, gamma, beta):
    x = x.astype(jnp.int32).reshape(B * L)
    e_sum = _sc_pool()(x, emb).reshape(B, EMB)
    return _tc_dense(e_sum, W_h, b_h, gamma, beta)


# speedup vs baseline: 1.0231x; 1.0031x over previous
import functools

import jax
import jax.numpy as jnp
from jax import lax
from jax.experimental import pallas as pl
from jax.experimental.pallas import tpu as pltpu
from jax.experimental.pallas import tpu_sc as plsc

VOCAB = 100000
EMB = 128
HID = 512
B = 4096
L = 200
EPS = 1e-5

CHUNK0 = 120
CHUNK1 = L - CHUNK0
VREGS = EMB // 16


@functools.lru_cache(maxsize=None)
def _sc_pool():
    info = plsc.get_sparse_core_info()
    nc, ns = info.num_cores, info.num_subcores
    nw = nc * ns
    spw = B // nw

    mesh = plsc.VectorSubcoreMesh(core_axis_name="c", subcore_axis_name="s")

    @functools.partial(
        pl.kernel,
        mesh=mesh,
        out_type=jax.ShapeDtypeStruct((B * EMB,), jnp.float32),
        scratch_types=[
            pltpu.VMEM((spw * L,), jnp.int32),
            pltpu.VMEM((3 * L, EMB), jnp.float32),
            pltpu.VMEM((spw * EMB,), jnp.float32),
            pltpu.SemaphoreType.DMA,
            pltpu.SemaphoreType.DMA,
            pltpu.SemaphoreType.DMA,
        ],
    )
    def pool(x_hbm, emb_hbm, out_hbm, idx_v, rows_v, out_v, sem0, sem1, sem2):
        c = lax.axis_index("c")
        s = lax.axis_index("s")
        wid = s * nc + c
        pltpu.sync_copy(x_hbm.at[pl.ds(wid * (spw * L), spw * L)], idx_v)

        def issue(i, slot, sem):
            pltpu.async_copy(
                emb_hbm.at[idx_v.at[pl.ds(i * L, CHUNK0)]],
                rows_v.at[pl.ds(slot * L, CHUNK0)], sem)
            pltpu.async_copy(
                emb_hbm.at[idx_v.at[pl.ds(i * L + CHUNK0, CHUNK1)]],
                rows_v.at[pl.ds(slot * L + CHUNK0, CHUNK1)], sem)

        def drain(i, slot, sem):
            pltpu.make_async_copy(
                emb_hbm.at[idx_v.at[pl.ds(i * L, CHUNK0)]],
                rows_v.at[pl.ds(slot * L, CHUNK0)], sem).wait()
            pltpu.make_async_copy(
                emb_hbm.at[idx_v.at[pl.ds(i * L + CHUNK0, CHUNK1)]],
                rows_v.at[pl.ds(slot * L + CHUNK0, CHUNK1)], sem).wait()

        def reduce_store(i, slot):
            UNROLL = 8

            def body(jj, acc):
                j0 = jj * UNROLL
                for u in range(UNROLL):
                    acc = tuple(
                        acc[k] + rows_v[slot * L + j0 + u, pl.ds(16 * k, 16)]
                        for k in range(VREGS))
                return acc

            acc = tuple(jnp.zeros((16,), jnp.float32) for _ in range(VREGS))
            acc = lax.fori_loop(0, L // UNROLL, body, acc)
            for k in range(VREGS):
                out_v[pl.ds(i * EMB + 16 * k, 16)] = acc[k]

        issue(0, 0, sem0)
        issue(1, 1, sem1)

        def triple(t, carry):
            i0 = 3 * t
            issue(i0 + 2, 2, sem2)
            drain(i0, 0, sem0)
            reduce_store(i0, 0)

            @pl.when(i0 + 3 < spw)
            def _():
                issue(i0 + 3, 0, sem0)

            drain(i0 + 1, 1, sem1)
            reduce_store(i0 + 1, 1)

            @pl.when(i0 + 4 < spw)
            def _():
                issue(i0 + 4, 1, sem1)

            drain(i0 + 2, 2, sem2)
            reduce_store(i0 + 2, 2)
            return carry

        ntrip = spw // 3
        lax.fori_loop(0, ntrip, triple, 0)
        rem = spw - 3 * ntrip
        epi = ((3 * ntrip, 0, sem0), (3 * ntrip + 1, 1, sem1),
               (3 * ntrip + 2, 2, sem2))[:rem]
        for i, slot, sem in epi:
            drain(i, slot, sem)
            reduce_store(i, slot)
        pltpu.sync_copy(out_v, out_hbm.at[pl.ds(wid * (spw * EMB), spw * EMB)])

    return pool


def _dense_body(e_ref, w_ref, b_ref, g_ref, bt_ref, out_ref):
    e = e_ref[...]
    w = w_ref[...]
    z = lax.dot_general(e, w, (((1,), (1,)), ((), ())),
                        preferred_element_type=jnp.float32)
    h = jax.nn.sigmoid(z * (1.0 / L) + b_ref[...])
    mu = jnp.mean(h, axis=0, keepdims=True)
    var = jnp.mean((h - mu) ** 2, axis=0, keepdims=True)
    out_ref[...] = (h - mu) * lax.rsqrt(var + EPS) * g_ref[...] + bt_ref[...]


def _tc_dense(e_sum, w_h, b_h, gamma, beta):
    return pl.pallas_call(
        _dense_body,
        out_shape=jax.ShapeDtypeStruct((B, HID), jnp.float32),
    )(e_sum, w_h, b_h.reshape(1, HID), gamma.reshape(1, HID),
      beta.reshape(1, HID))


def kernel(x, emb, W_h, b_h, gamma, beta):
    x = x.astype(jnp.int32).reshape(B * L)
    e_sum = _sc_pool()(x, emb).reshape(B, EMB)
    return _tc_dense(e_sum, W_h, b_h, gamma, beta)
